# baseline (speedup 1.0000x reference)
import jax
import jax.numpy as jnp
from jax import lax
from jax.experimental import pallas as pl
from jax.experimental.pallas import tpu as pltpu

M = 1000000
D = 64
B = 16384

CB = 32768
NB = 31
LG_CB = 15
CAP = 1024


def _merge_body(idx_ref, x_ref, u_ref, o_ref, st, lists, counts, ovfb, ovfc, ovfn):
    s = pl.program_id(0)

    @pl.when(s == 0)
    def _bucketize():
        def zero(w, _):
            counts[w] = 0
            return 0

        lax.fori_loop(0, NB + 1, zero, 0)
        ovfn[0] = 0

        def put(b, _):
            v = idx_ref[b]
            w = v >> LG_CB
            c = counts[w]
            slot = jnp.minimum(c, CAP)
            lists[w, slot] = ((v - (w << LG_CB)) << 14) | b
            oc = ovfn[0]
            spill = c >= CAP
            opos = jnp.where(spill, oc, B)
            ovfb[opos] = b
            ovfc[opos] = v
            ovfn[0] = oc + jnp.where(spill, 1, 0)
            counts[w] = c + 1
            return 0

        lax.fori_loop(0, B, put, 0)

    for t in range(CB // 128):
        st[t] = x_ref[:, t * 128:(t + 1) * 128]
    io = lax.broadcasted_iota(jnp.int32, (D, 128), 1)

    def apply(b, local):
        tt = local >> 7
        lane = local & 127
        utile = u_ref[b >> 7]
        rolled = pltpu.roll(utile, -(b & 127), 1)
        col = lax.broadcast_in_dim(rolled[:, 0:1], (D, 128), (0, 1))
        st[tt] = jnp.where(io == lane, col, st[tt])

    def from_list(j, _):
        e = lists[s, j]
        apply(e & 0x3FFF, e >> 14)
        return 0

    lax.fori_loop(0, jnp.minimum(counts[s], CAP), from_list, 0)

    def from_ovf(j, _):
        v = ovfc[j]

        @pl.when((v >> LG_CB) == s)
        def _():
            apply(ovfb[j], v - (s << LG_CB))

        return 0

    lax.fori_loop(0, ovfn[0], from_ovf, 0)

    for t in range(CB // 128):
        o_ref[:, t * 128:(t + 1) * 128] = st[t]


@jax.jit
def kernel(data, indices, updates):
    idx = indices.reshape(B)
    data_t = data.T
    upd3 = jnp.transpose(updates.reshape(128, 128, D), (0, 2, 1))

    out_t = pl.pallas_call(
        _merge_body,
        grid_spec=pltpu.PrefetchScalarGridSpec(
            num_scalar_prefetch=1,
            grid=(NB,),
            in_specs=[
                pl.BlockSpec((D, CB), lambda s, idx_ref: (0, s)),
                pl.BlockSpec((128, D, 128), lambda s, idx_ref: (0, 0, 0)),
            ],
            out_specs=pl.BlockSpec((D, CB), lambda s, idx_ref: (0, s)),
            scratch_shapes=[
                pltpu.VMEM((CB // 128, D, 128), jnp.float32),
                pltpu.SMEM((NB + 1, CAP + 1), jnp.int32),
                pltpu.SMEM((NB + 1,), jnp.int32),
                pltpu.SMEM((B + 1,), jnp.int32),
                pltpu.SMEM((B + 1,), jnp.int32),
                pltpu.SMEM((1,), jnp.int32),
            ],
        ),
        out_shape=jax.ShapeDtypeStruct((D, M), jnp.float32),
    )(idx, data_t, upd3)

    return out_t.T


# speedup vs baseline: 1.9201x; 1.9201x over previous
import jax
import jax.numpy as jnp
from jax.experimental import pallas as pl
from jax.experimental.pallas import tpu as pltpu

M = 1000000
D = 64
B = 16384

COPY_BLOCK = 8000
G = 32


def _copy_body(x_ref, o_ref):
    o_ref[...] = x_ref[...]


def _scatter_body(idx_ref, dst_any, upd_ref, out_any, sem):
    step = pl.program_id(0)
    del dst_any

    def issue(i, _):
        row = idx_ref[step * G + i]
        pltpu.make_async_copy(upd_ref.at[i], out_any.at[row], sem).start()
        return 0

    jax.lax.fori_loop(0, G, issue, 0)

    def drain(i, _):
        row = idx_ref[step * G + i]
        pltpu.make_async_copy(upd_ref.at[i], out_any.at[row], sem).wait()
        return 0

    jax.lax.fori_loop(0, G, drain, 0)


@jax.jit
def kernel(data, indices, updates):
    idx = indices.reshape(B)

    copied = pl.pallas_call(
        _copy_body,
        grid=(M // COPY_BLOCK,),
        in_specs=[pl.BlockSpec((COPY_BLOCK, D), lambda i: (i, 0))],
        out_specs=pl.BlockSpec((COPY_BLOCK, D), lambda i: (i, 0)),
        out_shape=jax.ShapeDtypeStruct((M, D), jnp.float32),
    )(data)

    out = pl.pallas_call(
        _scatter_body,
        grid_spec=pltpu.PrefetchScalarGridSpec(
            num_scalar_prefetch=1,
            grid=(B // G,),
            in_specs=[
                pl.BlockSpec(memory_space=pl.ANY),
                pl.BlockSpec((G, D), lambda s, idx_ref: (s, 0)),
            ],
            out_specs=pl.BlockSpec(memory_space=pl.ANY),
            scratch_shapes=[pltpu.SemaphoreType.DMA],
        ),
        out_shape=jax.ShapeDtypeStruct((M, D), jnp.float32),
        input_output_aliases={1: 0},
    )(idx, copied, updates)

    return out


# speedup vs baseline: 2.0579x; 1.0717x over previous
import jax
import jax.numpy as jnp
from jax import lax
from jax.experimental import pallas as pl
from jax.experimental.pallas import tpu as pltpu

M = 1000000
D = 64
B = 16384

CB = 32768
NB = 31
LG_CB = 15
CAP = 1024


def _merge_body(idx_ref, x_ref, u_ref, o_ref, st, lists, counts, ovfb, ovfc, ovfn):
    s = pl.program_id(0)

    @pl.when(s == 0)
    def _bucketize():
        def zero(w, _):
            counts[w] = 0
            return 0

        lax.fori_loop(0, NB + 1, zero, 0)
        ovfn[0] = 0

        def put(b, _):
            v = idx_ref[b]
            w = v >> LG_CB
            c = counts[w]
            slot = jnp.minimum(c, CAP)
            lists[w, slot] = ((v - (w << LG_CB)) << 14) | b
            oc = ovfn[0]
            spill = c >= CAP
            opos = jnp.where(spill, oc, B)
            ovfb[opos] = b
            ovfc[opos] = v
            ovfn[0] = oc + jnp.where(spill, 1, 0)
            counts[w] = c + 1
            return 0

        lax.fori_loop(0, B, put, 0, unroll=4)

    for t in range(CB // 128):
        st[t] = x_ref[:, t * 128:(t + 1) * 128]
    io = lax.broadcasted_iota(jnp.int32, (D, 128), 1)

    def apply(b, local):
        tt = local >> 7
        lane = local & 127
        utile = u_ref[b >> 7]
        rolled = pltpu.roll(utile, -(b & 127), 1)
        col = lax.broadcast_in_dim(rolled[:, 0:1], (D, 128), (0, 1))
        st[tt] = jnp.where(io == lane, col, st[tt])

    cnt = jnp.minimum(counts[s], CAP)

    def apply_pred(b, local, live):
        tt = local >> 7
        lane = jnp.where(live, local & 127, 128)
        utile = u_ref[b >> 7]
        rolled = pltpu.roll(utile, -(b & 127), 1)
        col = lax.broadcast_in_dim(rolled[:, 0:1], (D, 128), (0, 1))
        st[tt] = jnp.where(io == lane, col, st[tt])

    def from_list(j, _):
        live = j < cnt
        e = jnp.where(live, lists[s, jnp.minimum(j, CAP)], 0)
        apply_pred(e & 0x3FFF, e >> 14, live)
        return 0

    lax.fori_loop(0, CAP, from_list, 0, unroll=8)

    def from_ovf(j, _):
        v = ovfc[j]

        @pl.when((v >> LG_CB) == s)
        def _():
            apply(ovfb[j], v - (s << LG_CB))

        return 0

    lax.fori_loop(0, ovfn[0], from_ovf, 0)

    for t in range(CB // 128):
        o_ref[:, t * 128:(t + 1) * 128] = st[t]


@jax.jit
def kernel(data, indices, updates):
    idx = indices.reshape(B)
    data_t = data.T
    upd3 = jnp.transpose(updates.reshape(128, 128, D), (0, 2, 1))

    out_t = pl.pallas_call(
        _merge_body,
        grid_spec=pltpu.PrefetchScalarGridSpec(
            num_scalar_prefetch=1,
            grid=(NB,),
            in_specs=[
                pl.BlockSpec((D, CB), lambda s, idx_ref: (0, s)),
                pl.BlockSpec((128, D, 128), lambda s, idx_ref: (0, 0, 0)),
            ],
            out_specs=pl.BlockSpec((D, CB), lambda s, idx_ref: (0, s)),
            scratch_shapes=[
                pltpu.VMEM((CB // 128, D, 128), jnp.float32),
                pltpu.SMEM((NB + 1, CAP + 1), jnp.int32),
                pltpu.SMEM((NB + 1,), jnp.int32),
                pltpu.SMEM((B + 1,), jnp.int32),
                pltpu.SMEM((B + 1,), jnp.int32),
                pltpu.SMEM((1,), jnp.int32),
            ],
        ),
        out_shape=jax.ShapeDtypeStruct((D, M), jnp.float32),
    )(idx, data_t, upd3)

    return out_t.T
